# baseline (speedup 1.0000x reference)
import functools

import jax
import jax.numpy as jnp
from jax import lax
from jax.experimental import pallas as pl
from jax.experimental.pallas import tpu as pltpu
from jax.experimental.pallas import tpu_sc as plsc

_NUM_CORES = 2
_NUM_SUBCORES = 16
_NUM_WORKERS = _NUM_CORES * _NUM_SUBCORES

_ROWS = 1000000
_COLS = 64
_NUM_IDX = 4096
_IDX_PER_WORKER = _NUM_IDX // _NUM_WORKERS

_COPY_CHUNKS = 16
_COPY_CHUNK_ROWS = _ROWS // _COPY_CHUNKS


def _copy_body(x_hbm, y_hbm, *sems):
    for k in range(_COPY_CHUNKS):
        pltpu.make_async_copy(
            x_hbm.at[pl.ds(k * _COPY_CHUNK_ROWS, _COPY_CHUNK_ROWS)],
            y_hbm.at[pl.ds(k * _COPY_CHUNK_ROWS, _COPY_CHUNK_ROWS)],
            sems[k],
        ).start()
    for k in range(_COPY_CHUNKS):
        pltpu.make_async_copy(
            x_hbm.at[pl.ds(k * _COPY_CHUNK_ROWS, _COPY_CHUNK_ROWS)],
            y_hbm.at[pl.ds(k * _COPY_CHUNK_ROWS, _COPY_CHUNK_ROWS)],
            sems[k],
        ).wait()


_tc_copy = pl.pallas_call(
    _copy_body,
    out_shape=jax.ShapeDtypeStruct((_ROWS, _COLS), jnp.float32),
    in_specs=[pl.BlockSpec(memory_space=pltpu.MemorySpace.HBM)],
    out_specs=pl.BlockSpec(memory_space=pltpu.MemorySpace.HBM),
    scratch_shapes=[pltpu.SemaphoreType.DMA] * _COPY_CHUNKS,
)


@functools.partial(
    pl.kernel,
    mesh=plsc.VectorSubcoreMesh(
        core_axis_name="c", subcore_axis_name="s", num_cores=_NUM_CORES
    ),
    scratch_types=[
        pltpu.VMEM((_IDX_PER_WORKER,), jnp.int32),
        pltpu.VMEM((_COLS,), jnp.float32),
        pltpu.SemaphoreType.DMA,
    ],
    compiler_params=pltpu.CompilerParams(needs_layout_passes=False),
)
def _sc_fill(y_hbm, idx_hbm, idx_v, neg_v, sem):
    wid = lax.axis_index("s") * _NUM_CORES + lax.axis_index("c")
    base = wid * _IDX_PER_WORKER

    pltpu.sync_copy(idx_hbm.at[pl.ds(base, _IDX_PER_WORKER)], idx_v)

    neg16 = jnp.full((16,), -1.0, dtype=jnp.float32)
    for l in range(_COLS // 16):
        neg_v[pl.ds(l * 16, 16)] = neg16

    @pl.loop(0, _IDX_PER_WORKER // 16)
    def _(c):
        v = idx_v[pl.ds(c * 16, 16)]
        for j in range(16):
            u = jnp.take_along_axis(
                v, jnp.full((16,), j, jnp.int32), axis=0,
                mode="promise_in_bounds",
            )
            r = lax.reduce_max(u, axes=(0,))
            pltpu.async_copy(neg_v, y_hbm.at[r], sem)

    @pl.loop(0, _IDX_PER_WORKER)
    def _(i):
        pltpu.make_async_copy(neg_v, y_hbm.at[0], sem).wait()


def kernel(x, index):
    y = _tc_copy(x)
    y_ref = jax.new_ref(y)
    _sc_fill(y_ref, index)
    return jax.freeze(y_ref)


# speedup vs baseline: 15.8990x; 15.8990x over previous
import functools

import jax
import jax.numpy as jnp
from jax import lax
from jax.experimental import pallas as pl
from jax.experimental.pallas import tpu as pltpu
from jax.experimental.pallas import tpu_sc as plsc

_NUM_CORES = 2
_NUM_SUBCORES = 16
_NUM_WORKERS = _NUM_CORES * _NUM_SUBCORES

_ROWS = 1000000
_COLS = 64
_NUM_IDX = 4096
_IDX_PER_WORKER = _NUM_IDX // _NUM_WORKERS

_COPY_BLOCK_ROWS = 25000


def _copy_body(x_ref, o_ref):
    o_ref[...] = x_ref[...]


_tc_copy = pl.pallas_call(
    _copy_body,
    out_shape=jax.ShapeDtypeStruct((_ROWS, _COLS), jnp.float32),
    grid=(_ROWS // _COPY_BLOCK_ROWS,),
    in_specs=[pl.BlockSpec((_COPY_BLOCK_ROWS, _COLS), lambda i: (i, 0))],
    out_specs=pl.BlockSpec((_COPY_BLOCK_ROWS, _COLS), lambda i: (i, 0)),
)


@functools.partial(
    pl.kernel,
    mesh=plsc.VectorSubcoreMesh(
        core_axis_name="c", subcore_axis_name="s", num_cores=_NUM_CORES
    ),
    scratch_types=[
        pltpu.VMEM((_IDX_PER_WORKER,), jnp.int32),
        pltpu.VMEM((_COLS,), jnp.float32),
        pltpu.SemaphoreType.DMA,
    ],
    compiler_params=pltpu.CompilerParams(needs_layout_passes=False),
)
def _sc_fill(y_hbm, idx_hbm, idx_v, neg_v, sem):
    wid = lax.axis_index("s") * _NUM_CORES + lax.axis_index("c")
    base = wid * _IDX_PER_WORKER

    pltpu.sync_copy(idx_hbm.at[pl.ds(base, _IDX_PER_WORKER)], idx_v)

    neg16 = jnp.full((16,), -1.0, dtype=jnp.float32)
    for l in range(_COLS // 16):
        neg_v[pl.ds(l * 16, 16)] = neg16

    @pl.loop(0, _IDX_PER_WORKER // 16)
    def _(c):
        v = idx_v[pl.ds(c * 16, 16)]
        for j in range(16):
            u = jnp.take_along_axis(
                v, jnp.full((16,), j, jnp.int32), axis=0,
                mode="promise_in_bounds",
            )
            r = lax.reduce_max(u, axes=(0,))
            pltpu.async_copy(neg_v, y_hbm.at[r], sem)

    @pl.loop(0, _IDX_PER_WORKER)
    def _(i):
        pltpu.make_async_copy(neg_v, y_hbm.at[0], sem).wait()


def kernel(x, index):
    y = _tc_copy(x)
    y_ref = jax.new_ref(y)
    _sc_fill(y_ref, index)
    return jax.freeze(y_ref)
